# baseline (speedup 1.0000x reference)
import functools

import numpy as np
import jax
import jax.numpy as jnp
from jax import lax
from jax.experimental import pallas as pl
from jax.experimental.pallas import tpu as pltpu

ACTION_N = 1_000_000
LANES = 128

_A_BLOCK, _A_GRID, _A_OFF = 16384, 8, 0
_B_BLOCK, _B_GRID, _B_OFF = 124928, 6, 131072
_C_BLOCK, _C_GRID, _C_OFF = 15360, 8, 880640
_C_LEN = ACTION_N - _C_OFF

_CTR1 = np.arange(1, ACTION_N + 1, dtype=np.uint32)

_ROTS_A = (13, 15, 26, 6)
_ROTS_B = (17, 29, 16, 24)


def _threefry_bernoulli(x1, xv):
    ks = (0, 1, 0x1BD11BDA ^ 0 ^ 1)

    def rotl(v, r):
        return (v << jnp.uint32(r)) | (v >> jnp.uint32(32 - r))

    x0 = None
    for g in range(5):
        rots = _ROTS_A if g % 2 == 0 else _ROTS_B
        for r in rots:
            x0 = x1 if x0 is None else x0 + x1
            x1 = rotl(x1, r)
            x1 = x1 ^ x0
        k0 = ks[(g + 1) % 3]
        if k0:
            x0 = x0 + jnp.uint32(k0)
        x1 = x1 + jnp.uint32((ks[(g + 2) % 3] + g + 1) & 0xFFFFFFFF)

    bits = x0 ^ x1
    fbits = (bits >> jnp.uint32(9)) | jnp.uint32(0x3F800000)
    rand = lax.bitcast_convert_type(fbits, jnp.float32) - jnp.float32(1.0)
    return jnp.where(rand < xv, jnp.float32(1.0), jnp.float32(0.0))


def _stage_body(block, x_ref, c_ref, out_ref):
    rows = block // LANES
    xv = x_ref[...].reshape(rows, LANES)
    x1 = c_ref[...].reshape(rows, LANES)
    out_ref[...] = _threefry_bernoulli(x1, xv).reshape(block)


def _pipeline(x_hbm, c_hbm, out_hbm):
    for block, grid, off, ln in (
        (_A_BLOCK, _A_GRID, _A_OFF, _A_GRID * _A_BLOCK),
        (_B_BLOCK, _B_GRID, _B_OFF, _B_GRID * _B_BLOCK),
        (_C_BLOCK, _C_GRID, _C_OFF, _C_LEN),
    ):
        spec = pl.BlockSpec((block,), lambda i: (i,))
        pltpu.emit_pipeline(
            functools.partial(_stage_body, block),
            grid=(grid,),
            in_specs=(spec, spec),
            out_specs=(spec,),
        )(
            x_hbm.at[pl.ds(off, ln)],
            c_hbm.at[pl.ds(off, ln)],
            out_hbm.at[pl.ds(off, ln)],
        )


def kernel(x):
    return pl.pallas_call(
        _pipeline,
        out_shape=jax.ShapeDtypeStruct((ACTION_N,), jnp.float32),
        in_specs=[
            pl.BlockSpec(memory_space=pltpu.MemorySpace.HBM),
            pl.BlockSpec(memory_space=pltpu.MemorySpace.HBM),
        ],
        out_specs=pl.BlockSpec(memory_space=pltpu.MemorySpace.HBM),
    )(x, jnp.asarray(_CTR1))


# speedup vs baseline: 1.4757x; 1.4757x over previous
import numpy as np
import jax
import jax.numpy as jnp
from jax import lax
from jax.experimental import pallas as pl

ACTION_N = 1_000_000
LANES = 128
ROWS = 1120
BLOCK = ROWS * LANES
GRID = 7

_CTR1 = np.arange(1, GRID * BLOCK + 1, dtype=np.uint32)

_ROTS_A = (13, 15, 26, 6)
_ROTS_B = (17, 29, 16, 24)


def _threefry_bernoulli(x1, xv):
    ks = (0, 1, 0x1BD11BDA ^ 0 ^ 1)

    def rotl(v, r):
        return (v << jnp.uint32(r)) | (v >> jnp.uint32(32 - r))

    x0 = None
    for g in range(5):
        rots = _ROTS_A if g % 2 == 0 else _ROTS_B
        for r in rots:
            x0 = x1 if x0 is None else x0 + x1
            x1 = rotl(x1, r)
            x1 = x1 ^ x0
        k0 = ks[(g + 1) % 3]
        if k0:
            x0 = x0 + jnp.uint32(k0)
        x1 = x1 + jnp.uint32((ks[(g + 2) % 3] + g + 1) & 0xFFFFFFFF)

    bits = x0 ^ x1
    fbits = (bits >> jnp.uint32(9)) | jnp.uint32(0x3F800000)
    rand = lax.bitcast_convert_type(fbits, jnp.float32) - jnp.float32(1.0)
    return jnp.where(rand < xv, jnp.float32(1.0), jnp.float32(0.0))


def _bernoulli_block(x_ref, c_ref, out_ref):
    xv = x_ref[...].reshape(ROWS, LANES)
    x1 = c_ref[...].reshape(ROWS, LANES)
    out_ref[...] = _threefry_bernoulli(x1, xv).reshape(BLOCK)


def kernel(x):
    return pl.pallas_call(
        _bernoulli_block,
        out_shape=jax.ShapeDtypeStruct((ACTION_N,), jnp.float32),
        grid=(GRID,),
        in_specs=[
            pl.BlockSpec((BLOCK,), lambda i: (i,)),
            pl.BlockSpec((BLOCK,), lambda i: (i,)),
        ],
        out_specs=pl.BlockSpec((BLOCK,), lambda i: (i,)),
    )(x, jnp.asarray(_CTR1))


# speedup vs baseline: 1.4792x; 1.0023x over previous
import numpy as np
import jax
import jax.numpy as jnp
from jax import lax
from jax.experimental import pallas as pl
from jax.experimental.pallas import tpu as pltpu

ACTION_N = 1_000_000
LANES = 128
ROWS = 1120
BLOCK = ROWS * LANES
GRID = 7

_CTR1 = np.arange(1, GRID * BLOCK + 1, dtype=np.uint32)

_ROTS_A = (13, 15, 26, 6)
_ROTS_B = (17, 29, 16, 24)


def _threefry_bernoulli(x1, xv):
    ks = (0, 1, 0x1BD11BDA ^ 0 ^ 1)

    def rotl(v, r):
        return (v << jnp.uint32(r)) | (v >> jnp.uint32(32 - r))

    x0 = None
    for g in range(5):
        rots = _ROTS_A if g % 2 == 0 else _ROTS_B
        for r in rots:
            x0 = x1 if x0 is None else x0 + x1
            x1 = rotl(x1, r)
            x1 = x1 ^ x0
        k0 = ks[(g + 1) % 3]
        if k0:
            x0 = x0 + jnp.uint32(k0)
        x1 = x1 + jnp.uint32((ks[(g + 2) % 3] + g + 1) & 0xFFFFFFFF)

    bits = x0 ^ x1
    fbits = (bits >> jnp.uint32(9)) | jnp.uint32(0x3F800000)
    rand = lax.bitcast_convert_type(fbits, jnp.float32) - jnp.float32(1.0)
    return jnp.where(rand < xv, jnp.float32(1.0), jnp.float32(0.0))


def _bernoulli_block(x_ref, c_ref, out_ref):
    xv = x_ref[...].reshape(ROWS, LANES)
    x1 = c_ref[...].reshape(ROWS, LANES)
    out_ref[...] = _threefry_bernoulli(x1, xv).reshape(BLOCK)


def kernel(x):
    return pl.pallas_call(
        _bernoulli_block,
        out_shape=jax.ShapeDtypeStruct((ACTION_N,), jnp.float32),
        grid=(GRID,),
        in_specs=[
            pl.BlockSpec((BLOCK,), lambda i: (i,)),
            pl.BlockSpec((BLOCK,), lambda i: (i,)),
        ],
        out_specs=pl.BlockSpec((BLOCK,), lambda i: (i,)),
        compiler_params=pltpu.CompilerParams(
            dimension_semantics=("arbitrary",)),
    )(x, jnp.asarray(_CTR1))
